# baseline (speedup 1.0000x reference)
import functools

import jax
import jax.numpy as jnp
from jax import lax
from jax.experimental import pallas as pl
from jax.experimental.pallas import tpu as pltpu
from jax.experimental.pallas import tpu_sc as plsc

_IBIG = 1 << 20
_NC, _NS, _L = 2, 16, 16
_NW = _NC * _NS
_RPC = 2
_BB = 64


def _sc_select(nop_ref, pt_ref, tbu_ref, out_ref, ptb, nopb, tbub, actb,
               *, rows, n_jobs, n_mas, n_trs, n_ops):
    cid = lax.axis_index("c")
    sid = lax.axis_index("s")
    wid = sid * _NC + cid
    base = wid * rows
    n_jc = nopb.shape[1] // _L
    lane = lax.iota(jnp.int32, _L)

    def chunk_body(c, carry):
        r0 = base + c * _RPC
        pltpu.sync_copy(pt_ref.at[pl.ds(r0, _RPC)], ptb)
        pltpu.sync_copy(nop_ref.at[pl.ds(r0, _RPC)], nopb)
        pltpu.sync_copy(tbu_ref.at[pl.ds(r0, _RPC)], tbub)
        for r in range(_RPC):
            curval = jnp.full((_L,), jnp.inf, jnp.float32)
            curkey = jnp.full((_L,), _IBIG, jnp.int32)
            for jc in range(n_jc):
                idx16 = nopb[r, pl.ds(jc * _L, _L)]
                jkey = (jc * _L + lane) * n_mas
                pad = n_jobs - jc * _L
                rvec = jnp.full((_L,), r, jnp.int32)
                for m in range(n_mas):
                    v = plsc.load_gather(ptb, [rvec, idx16 + m * n_ops])
                    better = (v < curval) & (v != 0.0)
                    if pad < _L:
                        better = better & (lane < pad)
                    curval = jnp.where(better, v, curval)
                    curkey = jnp.where(better, jkey + m, curkey)
            minv = jnp.min(curval)
            fkey = jnp.min(jnp.where(curval == minv, curkey, _IBIG))
            fkey = jnp.where(minv == jnp.inf, 0, fkey)
            tv = tbub[r]
            tkey = jnp.min(jnp.where(tv == jnp.min(tv), lane, _L))
            act = 1 + fkey * n_trs + tkey
            actb[r] = jnp.full((_L,), act, jnp.int32)
        pltpu.sync_copy(actb, out_ref.at[pl.ds(r0, _RPC)])
        return carry

    lax.fori_loop(0, rows // _RPC, chunk_body, 0)


def _tc_onehot(act_ref, out_ref):
    act = act_ref[:, :1]
    n_cols = out_ref.shape[1]
    col = lax.broadcasted_iota(jnp.int32, (act_ref.shape[0], n_cols), 1)
    out_ref[...] = jnp.where(col == act, 1.0, 0.0).astype(jnp.float32)


def kernel(job_done, machine_busy_until, truck_location, next_op, proc_times,
           truck_busy_until, action_mask):
    B, n_jobs = job_done.shape
    n_mas = machine_busy_until.shape[1]
    n_trs = truck_location.shape[1]
    n_ops = proc_times.shape[2]
    n_cols = 1 + n_jobs * n_mas * n_trs
    rows = B // _NW

    jpad = (-n_jobs) % _L
    nop_p = jnp.pad(next_op, ((0, 0), (0, jpad)))
    tbu_p = jnp.pad(truck_busy_until, ((0, 0), (0, _L - n_trs)),
                    constant_values=jnp.inf)
    pt2 = proc_times.reshape(B, n_mas * n_ops)

    sel = functools.partial(_sc_select, rows=rows, n_jobs=n_jobs,
                            n_mas=n_mas, n_trs=n_trs, n_ops=n_ops)
    act16 = pl.kernel(
        sel,
        out_type=jax.ShapeDtypeStruct((B, _L), jnp.int32),
        mesh=plsc.VectorSubcoreMesh(core_axis_name="c", subcore_axis_name="s",
                                    num_cores=_NC, num_subcores=_NS),
        compiler_params=pltpu.CompilerParams(needs_layout_passes=False),
        scratch_types=[
            pltpu.VMEM((_RPC, n_mas * n_ops), jnp.float32),
            pltpu.VMEM((_RPC, n_jobs + jpad), jnp.int32),
            pltpu.VMEM((_RPC, _L), jnp.float32),
            pltpu.VMEM((_RPC, _L), jnp.int32),
        ],
    )(nop_p, pt2, tbu_p)

    logits = pl.pallas_call(
        _tc_onehot,
        grid=(B // _BB,),
        in_specs=[pl.BlockSpec((_BB, _L), lambda i: (i, 0))],
        out_specs=pl.BlockSpec((_BB, n_cols), lambda i: (i, 0)),
        out_shape=jax.ShapeDtypeStruct((B, n_cols), jnp.float32),
    )(act16)
    return (logits, action_mask)
